# Initial kernel scaffold; baseline (speedup 1.0000x reference)
import functools

import jax
import jax.numpy as jnp
from jax import lax
from jax.experimental import pallas as pl
from jax.experimental.pallas import tpu as pltpu
from jax.experimental.pallas import tpu_sc as plsc

N_CORES = 2
N_SUBCORES = 16
N_WORKERS = N_CORES * N_SUBCORES
BATCH = 128


def _make_seg_sum(n_pad, d, nblocks, do_count):
    mesh = plsc.VectorSubcoreMesh(core_axis_name="c", subcore_axis_name="s")
    rows_per_sub = n_pad // N_SUBCORES

    out_type = [jax.ShapeDtypeStruct((N_CORES, n_pad, d), jnp.float32)]
    scratch = [
        pltpu.VMEM((nblocks, BATCH), jnp.int32),
        pltpu.VMEM((nblocks, BATCH), jnp.int32),
        pltpu.VMEM((BATCH, d), jnp.float32),
        pltpu.VMEM_SHARED((n_pad, d), jnp.float32),
        pltpu.SemaphoreType.DMA,
    ]
    if do_count:
        out_type.append(jax.ShapeDtypeStruct((N_CORES, n_pad, 16), jnp.float32))
        scratch += [
            pltpu.VMEM((BATCH, 16), jnp.float32),
            pltpu.VMEM_SHARED((n_pad, 16), jnp.float32),
        ]

    @functools.partial(
        pl.kernel, out_type=out_type, mesh=mesh, scratch_types=scratch
    )
    def seg_sum(y_hbm, src_hbm, dst_hbm, *refs):
        if do_count:
            (out_hbm, cnt_hbm, src_v, dst_v, rows_v, acc, sem,
             ones_v, cnt_acc) = refs
        else:
            out_hbm, src_v, dst_v, rows_v, acc, sem = refs
        c = lax.axis_index("c")
        s = lax.axis_index("s")
        wid = s * N_CORES + c

        zero16 = jnp.zeros((1, 16), jnp.float32)

        @pl.loop(0, BATCH)
        def _(r):
            @pl.loop(0, d, step=16)
            def _(col):
                rows_v.at[pl.ds(r, 1), pl.ds(col, 16)][...] = zero16

        if do_count:
            @pl.loop(0, BATCH)
            def _(r):
                ones_v.at[pl.ds(r, 1), pl.ds(0, 16)][...] = zero16

        base = s * rows_per_sub
        nfull = rows_per_sub // BATCH
        rem = rows_per_sub % BATCH

        @pl.loop(0, nfull)
        def _(i):
            pltpu.sync_copy(rows_v, acc.at[pl.ds(base + i * BATCH, BATCH)])
            if do_count:
                pltpu.sync_copy(
                    ones_v, cnt_acc.at[pl.ds(base + i * BATCH, BATCH)])

        if rem:
            pltpu.sync_copy(rows_v.at[pl.ds(0, rem)],
                            acc.at[pl.ds(base + nfull * BATCH, rem)])
            if do_count:
                pltpu.sync_copy(ones_v.at[pl.ds(0, rem)],
                                cnt_acc.at[pl.ds(base + nfull * BATCH, rem)])

        if do_count:
            one16 = jnp.ones((1, 16), jnp.float32)

            @pl.loop(0, BATCH)
            def _(r):
                ones_v.at[pl.ds(r, 1), pl.ds(0, 16)][...] = one16

        plsc.subcore_barrier()

        pltpu.sync_copy(src_hbm.at[wid], src_v)
        pltpu.sync_copy(dst_hbm.at[wid], dst_v)

        @pl.loop(0, nblocks)
        def _(j):
            pltpu.async_copy(y_hbm.at[src_v.at[j]], rows_v, sem).wait()
            pltpu.sync_copy(rows_v, acc.at[dst_v.at[j]], add=True)
            if do_count:
                pltpu.sync_copy(ones_v, cnt_acc.at[dst_v.at[j]], add=True)

        plsc.subcore_barrier()

        pltpu.sync_copy(acc.at[pl.ds(base, rows_per_sub)],
                        out_hbm.at[c, pl.ds(base, rows_per_sub)])
        if do_count:
            pltpu.sync_copy(cnt_acc.at[pl.ds(base, rows_per_sub)],
                            cnt_hbm.at[c, pl.ds(base, rows_per_sub)])

    return seg_sum


_ROW_BLK = 400


def _mm2_kernel(x_ref, wa_ref, wb_ref, a_ref, b_ref):
    xb = x_ref[...]
    a_ref[...] = jnp.dot(xb, wa_ref[...], preferred_element_type=jnp.float32)
    b_ref[...] = jnp.dot(xb, wb_ref[...], preferred_element_type=jnp.float32)


def _mm2(x, wa_t, wb_t):
    n, d = x.shape
    grid = n // _ROW_BLK
    w_spec = pl.BlockSpec((d, wa_t.shape[1]), lambda i: (0, 0))
    row_spec = pl.BlockSpec((_ROW_BLK, d), lambda i: (i, 0))
    out_spec = pl.BlockSpec((_ROW_BLK, wa_t.shape[1]), lambda i: (i, 0))
    return pl.pallas_call(
        _mm2_kernel,
        grid=(grid,),
        in_specs=[row_spec, w_spec, w_spec],
        out_specs=[out_spec, out_spec],
        out_shape=[
            jax.ShapeDtypeStruct((n, wa_t.shape[1]), jnp.float32),
            jax.ShapeDtypeStruct((n, wb_t.shape[1]), jnp.float32),
        ],
    )(x, wa_t, wb_t)


def _combine_mm2_kernel(aggp_ref, cntp_ref, r_ref, bl_ref, wa_ref, wb_ref,
                        a_ref, b_ref):
    agg = aggp_ref[0] + aggp_ref[1]
    cnt = cntp_ref[0, :, 0:1] + cntp_ref[1, :, 0:1]
    mean = agg / jnp.maximum(cnt, 1.0)
    h = jnp.maximum(mean + bl_ref[...] + r_ref[...], 0.0)
    a_ref[...] = jnp.dot(h, wa_ref[...], preferred_element_type=jnp.float32)
    b_ref[...] = jnp.dot(h, wb_ref[...], preferred_element_type=jnp.float32)


def _combine_mm2(aggp, cntp, r, bl, wa_t, wb_t):
    n, d = r.shape
    grid = n // _ROW_BLK
    return pl.pallas_call(
        _combine_mm2_kernel,
        grid=(grid,),
        in_specs=[
            pl.BlockSpec((N_CORES, _ROW_BLK, d), lambda i: (0, i, 0)),
            pl.BlockSpec((N_CORES, _ROW_BLK, 16), lambda i: (0, i, 0)),
            pl.BlockSpec((_ROW_BLK, d), lambda i: (i, 0)),
            pl.BlockSpec((d,), lambda i: (0,)),
            pl.BlockSpec((d, wa_t.shape[1]), lambda i: (0, 0)),
            pl.BlockSpec((d, wb_t.shape[1]), lambda i: (0, 0)),
        ],
        out_specs=[
            pl.BlockSpec((_ROW_BLK, wa_t.shape[1]), lambda i: (i, 0)),
            pl.BlockSpec((_ROW_BLK, wb_t.shape[1]), lambda i: (i, 0)),
        ],
        out_shape=[
            jax.ShapeDtypeStruct((n, wa_t.shape[1]), jnp.float32),
            jax.ShapeDtypeStruct((n, wb_t.shape[1]), jnp.float32),
        ],
    )(aggp, cntp, r, bl, wa_t, wb_t)


def _combine_out_kernel(aggp_ref, cntp_ref, r_ref, bl_ref, w_ref, blin_ref,
                        o_ref):
    agg = aggp_ref[0] + aggp_ref[1]
    cnt = cntp_ref[0, :, 0:1] + cntp_ref[1, :, 0:1]
    mean = agg / jnp.maximum(cnt, 1.0)
    h = jnp.maximum(mean + bl_ref[...] + r_ref[...], 0.0)
    o_ref[...] = (
        jnp.dot(h, w_ref[...], preferred_element_type=jnp.float32)
        + blin_ref[...]
    )


def _combine_out(aggp, cntp, r, bl, w_t, blin):
    n, d = r.shape
    grid = n // _ROW_BLK
    return pl.pallas_call(
        _combine_out_kernel,
        grid=(grid,),
        in_specs=[
            pl.BlockSpec((N_CORES, _ROW_BLK, d), lambda i: (0, i, 0)),
            pl.BlockSpec((N_CORES, _ROW_BLK, 16), lambda i: (0, i, 0)),
            pl.BlockSpec((_ROW_BLK, d), lambda i: (i, 0)),
            pl.BlockSpec((d,), lambda i: (0,)),
            pl.BlockSpec((d, w_t.shape[1]), lambda i: (0, 0)),
            pl.BlockSpec((w_t.shape[1],), lambda i: (0,)),
        ],
        out_specs=pl.BlockSpec((_ROW_BLK, w_t.shape[1]), lambda i: (i, 0)),
        out_shape=jax.ShapeDtypeStruct((n, w_t.shape[1]), jnp.float32),
    )(aggp, cntp, r, bl, w_t, blin)


def kernel(x, edge_index, Wl1, bl1, Wr1, Wl2, bl2, Wr2, Wlin, blin):
    n, d = x.shape
    e = edge_index.shape[1]

    nblocks = -(-e // (N_WORKERS * BATCH))
    e_pad = N_WORKERS * nblocks * BATCH
    n_pad = -(-(n + 1) // N_SUBCORES) * N_SUBCORES

    pad = e_pad - e
    src = jnp.concatenate(
        [edge_index[0], jnp.zeros((pad,), jnp.int32)]
    ).reshape(N_WORKERS, nblocks, BATCH)
    dst = jnp.concatenate(
        [edge_index[1], jnp.full((pad,), n, jnp.int32)]
    ).reshape(N_WORKERS, nblocks, BATCH)

    seg_sum_cnt = _make_seg_sum(n_pad, d, nblocks, do_count=True)
    seg_sum = _make_seg_sum(n_pad, d, nblocks, do_count=False)

    y1, r1 = _mm2(x, Wl1.T, Wr1.T)
    aggp1, cntp = seg_sum_cnt(y1, src, dst)
    y2, r2 = _combine_mm2(aggp1, cntp, r1, bl1, Wl2.T, Wr2.T)
    aggp2 = seg_sum(y2, src, dst)
    return _combine_out(aggp2, cntp, r2, bl2, Wlin.T, blin)


# speedup vs baseline: 3.1049x; 3.1049x over previous
import functools

import jax
import jax.numpy as jnp
from jax import lax
from jax.experimental import pallas as pl
from jax.experimental.pallas import tpu as pltpu
from jax.experimental.pallas import tpu_sc as plsc

N_CORES = 2
N_SUBCORES = 16
N_WORKERS = N_CORES * N_SUBCORES
BATCH = 128
IDX_CHUNK = 16


def _fill(ref, nrows, d, value16):
    @pl.loop(0, nrows)
    def _(r):
        @pl.loop(0, d, step=16)
        def _(col):
            ref.at[pl.ds(r, 1), pl.ds(col, 16)][...] = value16


def _zero_acc_slice(zsrc, acc, base, rows_per_sub):
    nfull = rows_per_sub // BATCH
    rem = rows_per_sub % BATCH

    @pl.loop(0, nfull)
    def _(i):
        pltpu.sync_copy(zsrc, acc.at[pl.ds(base + i * BATCH, BATCH)])

    if rem:
        pltpu.sync_copy(zsrc.at[pl.ds(0, rem)],
                        acc.at[pl.ds(base + nfull * BATCH, rem)])


def _make_seg_sum(n_pad, d, nblocks):
    mesh = plsc.VectorSubcoreMesh(core_axis_name="c", subcore_axis_name="s")
    rows_per_sub = n_pad // N_SUBCORES
    nchunks = nblocks // IDX_CHUNK
    scratch = [
        pltpu.VMEM((IDX_CHUNK, BATCH), jnp.int32),
        pltpu.VMEM((IDX_CHUNK, BATCH), jnp.int32),
        pltpu.VMEM((BATCH, d), jnp.float32),
        pltpu.VMEM_SHARED((n_pad, d), jnp.float32),
        pltpu.SemaphoreType.DMA,
    ]

    @functools.partial(
        pl.kernel,
        out_type=jax.ShapeDtypeStruct((N_CORES, n_pad, d), jnp.float32),
        mesh=mesh,
        scratch_types=scratch,
    )
    def seg_sum(y_hbm, src_hbm, dst_hbm, out_hbm, src_v, dst_v, rows_v, acc,
                sem):
        c = lax.axis_index("c")
        s = lax.axis_index("s")
        wid = s * N_CORES + c

        _fill(rows_v, BATCH, d, jnp.zeros((1, 16), jnp.float32))
        base = s * rows_per_sub
        _zero_acc_slice(rows_v, acc, base, rows_per_sub)
        plsc.subcore_barrier()

        @pl.loop(0, nchunks)
        def _(ci):
            pltpu.sync_copy(src_hbm.at[wid, pl.ds(ci * IDX_CHUNK, IDX_CHUNK)],
                            src_v)
            pltpu.sync_copy(dst_hbm.at[wid, pl.ds(ci * IDX_CHUNK, IDX_CHUNK)],
                            dst_v)

            @pl.loop(0, IDX_CHUNK)
            def _(j):
                pltpu.async_copy(y_hbm.at[src_v.at[j]], rows_v, sem).wait()
                pltpu.sync_copy(rows_v, acc.at[dst_v.at[j]], add=True)

        plsc.subcore_barrier()

        pltpu.sync_copy(acc.at[pl.ds(base, rows_per_sub)],
                        out_hbm.at[c, pl.ds(base, rows_per_sub)])

    return seg_sum


def _make_count(n_pad, nblocks):
    mesh = plsc.VectorSubcoreMesh(core_axis_name="c", subcore_axis_name="s")
    rows_per_sub = n_pad // N_SUBCORES
    nchunks = nblocks // IDX_CHUNK
    scratch = [
        pltpu.VMEM((IDX_CHUNK, BATCH), jnp.int32),
        pltpu.VMEM((BATCH, 128), jnp.float32),
        pltpu.VMEM_SHARED((n_pad, 128), jnp.float32),
    ]

    @functools.partial(
        pl.kernel,
        out_type=jax.ShapeDtypeStruct((N_CORES, n_pad, 128), jnp.float32),
        mesh=mesh,
        scratch_types=scratch,
    )
    def count(dst_hbm, out_hbm, dst_v, ones_v, acc):
        c = lax.axis_index("c")
        s = lax.axis_index("s")
        wid = s * N_CORES + c

        _fill(ones_v, BATCH, 128, jnp.zeros((1, 16), jnp.float32))
        base = s * rows_per_sub
        _zero_acc_slice(ones_v, acc, base, rows_per_sub)
        _fill(ones_v, BATCH, 128, jnp.ones((1, 16), jnp.float32))
        plsc.subcore_barrier()

        @pl.loop(0, nchunks)
        def _(ci):
            pltpu.sync_copy(dst_hbm.at[wid, pl.ds(ci * IDX_CHUNK, IDX_CHUNK)],
                            dst_v)

            @pl.loop(0, IDX_CHUNK)
            def _(j):
                pltpu.sync_copy(ones_v, acc.at[dst_v.at[j]], add=True)

        plsc.subcore_barrier()
        pltpu.sync_copy(acc.at[pl.ds(base, rows_per_sub)],
                        out_hbm.at[c, pl.ds(base, rows_per_sub)])

    return count


_ROW_BLK = 400


def _mm2_kernel(x_ref, wa_ref, wb_ref, a_ref, b_ref):
    xb = x_ref[...]
    a_ref[...] = jnp.dot(xb, wa_ref[...], preferred_element_type=jnp.float32)
    b_ref[...] = jnp.dot(xb, wb_ref[...], preferred_element_type=jnp.float32)


def _mm2(x, wa_t, wb_t):
    n, d = x.shape
    grid = n // _ROW_BLK
    w_spec = pl.BlockSpec((d, wa_t.shape[1]), lambda i: (0, 0))
    row_spec = pl.BlockSpec((_ROW_BLK, d), lambda i: (i, 0))
    out_spec = pl.BlockSpec((_ROW_BLK, wa_t.shape[1]), lambda i: (i, 0))
    return pl.pallas_call(
        _mm2_kernel,
        grid=(grid,),
        in_specs=[row_spec, w_spec, w_spec],
        out_specs=[out_spec, out_spec],
        out_shape=[
            jax.ShapeDtypeStruct((n, wa_t.shape[1]), jnp.float32),
            jax.ShapeDtypeStruct((n, wb_t.shape[1]), jnp.float32),
        ],
    )(x, wa_t, wb_t)


def _combine_mm2_kernel(aggp_ref, cntp_ref, r_ref, bl_ref, wa_ref, wb_ref,
                        a_ref, b_ref):
    agg = aggp_ref[0] + aggp_ref[1]
    cnt = cntp_ref[0, :, 0:1] + cntp_ref[1, :, 0:1]
    mean = agg / jnp.maximum(cnt, 1.0)
    h = jnp.maximum(mean + bl_ref[...] + r_ref[...], 0.0)
    a_ref[...] = jnp.dot(h, wa_ref[...], preferred_element_type=jnp.float32)
    b_ref[...] = jnp.dot(h, wb_ref[...], preferred_element_type=jnp.float32)


def _combine_mm2(aggp, cntp, r, bl, wa_t, wb_t):
    n, d = r.shape
    grid = n // _ROW_BLK
    return pl.pallas_call(
        _combine_mm2_kernel,
        grid=(grid,),
        in_specs=[
            pl.BlockSpec((N_CORES, _ROW_BLK, d), lambda i: (0, i, 0)),
            pl.BlockSpec((N_CORES, _ROW_BLK, 128), lambda i: (0, i, 0)),
            pl.BlockSpec((_ROW_BLK, d), lambda i: (i, 0)),
            pl.BlockSpec((d,), lambda i: (0,)),
            pl.BlockSpec((d, wa_t.shape[1]), lambda i: (0, 0)),
            pl.BlockSpec((d, wb_t.shape[1]), lambda i: (0, 0)),
        ],
        out_specs=[
            pl.BlockSpec((_ROW_BLK, wa_t.shape[1]), lambda i: (i, 0)),
            pl.BlockSpec((_ROW_BLK, wb_t.shape[1]), lambda i: (i, 0)),
        ],
        out_shape=[
            jax.ShapeDtypeStruct((n, wa_t.shape[1]), jnp.float32),
            jax.ShapeDtypeStruct((n, wb_t.shape[1]), jnp.float32),
        ],
    )(aggp, cntp, r, bl, wa_t, wb_t)


def _combine_out_kernel(aggp_ref, cntp_ref, r_ref, bl_ref, w_ref, blin_ref,
                        o_ref):
    agg = aggp_ref[0] + aggp_ref[1]
    cnt = cntp_ref[0, :, 0:1] + cntp_ref[1, :, 0:1]
    mean = agg / jnp.maximum(cnt, 1.0)
    h = jnp.maximum(mean + bl_ref[...] + r_ref[...], 0.0)
    o_ref[...] = (
        jnp.dot(h, w_ref[...], preferred_element_type=jnp.float32)
        + blin_ref[...]
    )


def _combine_out(aggp, cntp, r, bl, w_t, blin):
    n, d = r.shape
    grid = n // _ROW_BLK
    return pl.pallas_call(
        _combine_out_kernel,
        grid=(grid,),
        in_specs=[
            pl.BlockSpec((N_CORES, _ROW_BLK, d), lambda i: (0, i, 0)),
            pl.BlockSpec((N_CORES, _ROW_BLK, 128), lambda i: (0, i, 0)),
            pl.BlockSpec((_ROW_BLK, d), lambda i: (i, 0)),
            pl.BlockSpec((d,), lambda i: (0,)),
            pl.BlockSpec((d, w_t.shape[1]), lambda i: (0, 0)),
            pl.BlockSpec((w_t.shape[1],), lambda i: (0,)),
        ],
        out_specs=pl.BlockSpec((_ROW_BLK, w_t.shape[1]), lambda i: (i, 0)),
        out_shape=jax.ShapeDtypeStruct((n, w_t.shape[1]), jnp.float32),
    )(aggp, cntp, r, bl, w_t, blin)


def kernel(x, edge_index, Wl1, bl1, Wr1, Wl2, bl2, Wr2, Wlin, blin):
    n, d = x.shape
    e = edge_index.shape[1]

    nblocks = -(-e // (N_WORKERS * BATCH * IDX_CHUNK)) * IDX_CHUNK
    e_pad = N_WORKERS * nblocks * BATCH
    n_pad = -(-(n + 1) // (N_SUBCORES * 8)) * (N_SUBCORES * 8)

    pad = e_pad - e
    src = jnp.concatenate(
        [edge_index[0], jnp.zeros((pad,), jnp.int32)]
    ).reshape(N_WORKERS, nblocks, BATCH)
    dst = jnp.concatenate(
        [edge_index[1], jnp.full((pad,), n, jnp.int32)]
    ).reshape(N_WORKERS, nblocks, BATCH)

    seg_sum = _make_seg_sum(n_pad, d, nblocks)
    count = _make_count(n_pad, nblocks)

    cntp = count(dst)
    y1, r1 = _mm2(x, Wl1.T, Wr1.T)
    aggp1 = seg_sum(y1, src, dst)
    y2, r2 = _combine_mm2(aggp1, cntp, r1, bl1, Wl2.T, Wr2.T)
    aggp2 = seg_sum(y2, src, dst)
    return _combine_out(aggp2, cntp, r2, bl2, Wlin.T, blin)


# speedup vs baseline: 3.5213x; 1.1341x over previous
import functools

import jax
import jax.numpy as jnp
from jax import lax
from jax.experimental import pallas as pl
from jax.experimental.pallas import tpu as pltpu
from jax.experimental.pallas import tpu_sc as plsc

N_CORES = 2
N_SUBCORES = 16
N_WORKERS = N_CORES * N_SUBCORES
BATCH = 128
IDX_CHUNK = 16


def _fill(ref, nrows, d, value16):
    @pl.loop(0, nrows)
    def _(r):
        @pl.loop(0, d, step=16)
        def _(col):
            ref.at[pl.ds(r, 1), pl.ds(col, 16)][...] = value16


def _zero_acc_slice(zsrc, acc, base, rows_per_sub):
    nfull = rows_per_sub // BATCH
    rem = rows_per_sub % BATCH

    @pl.loop(0, nfull)
    def _(i):
        pltpu.sync_copy(zsrc, acc.at[pl.ds(base + i * BATCH, BATCH)])

    if rem:
        pltpu.sync_copy(zsrc.at[pl.ds(0, rem)],
                        acc.at[pl.ds(base + nfull * BATCH, rem)])


def _make_seg_sum(n_pad, d, nblocks):
    mesh = plsc.VectorSubcoreMesh(core_axis_name="c", subcore_axis_name="s")
    rows_per_sub = n_pad // N_SUBCORES
    nchunks = nblocks // IDX_CHUNK
    scratch = [
        pltpu.VMEM((IDX_CHUNK, BATCH), jnp.int32),
        pltpu.VMEM((IDX_CHUNK, BATCH), jnp.int32),
        pltpu.VMEM((BATCH, d), jnp.float32),
        pltpu.VMEM((BATCH, d), jnp.float32),
        pltpu.VMEM_SHARED((n_pad, d), jnp.float32),
        pltpu.SemaphoreType.DMA,
        pltpu.SemaphoreType.DMA,
        pltpu.SemaphoreType.DMA,
        pltpu.SemaphoreType.DMA,
    ]

    @functools.partial(
        pl.kernel,
        out_type=jax.ShapeDtypeStruct((N_CORES, n_pad, d), jnp.float32),
        mesh=mesh,
        scratch_types=scratch,
    )
    def seg_sum(y_hbm, src_hbm, dst_hbm, out_hbm, src_v, dst_v, rows_a,
                rows_b, acc, sem_ga, sem_gb, sem_sa, sem_sb):
        c = lax.axis_index("c")
        s = lax.axis_index("s")
        wid = s * N_CORES + c

        def gather(j, buf, sem):
            return pltpu.async_copy(y_hbm.at[src_v.at[j]], buf, sem)

        def scatter(j, buf, sem):
            return pltpu.async_copy(buf, acc.at[dst_v.at[j]], sem, add=True)

        def wait_gather(buf, sem):
            pltpu.make_async_copy(y_hbm.at[src_v.at[0]], buf, sem).wait()

        def wait_scatter(buf, sem):
            pltpu.make_async_copy(buf, acc.at[dst_v.at[0]], sem).wait()

        _fill(rows_a, BATCH, d, jnp.zeros((1, 16), jnp.float32))
        base = s * rows_per_sub
        _zero_acc_slice(rows_a, acc, base, rows_per_sub)
        plsc.subcore_barrier()

        @pl.loop(0, nchunks)
        def _(ci):
            pltpu.sync_copy(src_hbm.at[wid, pl.ds(ci * IDX_CHUNK, IDX_CHUNK)],
                            src_v)
            pltpu.sync_copy(dst_hbm.at[wid, pl.ds(ci * IDX_CHUNK, IDX_CHUNK)],
                            dst_v)
            gather(0, rows_a, sem_ga)

            @pl.loop(0, IDX_CHUNK // 2)
            def _(p):
                j = 2 * p
                wait_gather(rows_a, sem_ga)
                d_sa = scatter(j, rows_a, sem_sa)

                @pl.when(p > 0)
                def _():
                    wait_scatter(rows_b, sem_sb)

                d_gb = gather(j + 1, rows_b, sem_gb)
                d_gb.wait()
                scatter(j + 1, rows_b, sem_sb)
                d_sa.wait()

                @pl.when(p < IDX_CHUNK // 2 - 1)
                def _():
                    gather(j + 2, rows_a, sem_ga)

            wait_scatter(rows_b, sem_sb)

        plsc.subcore_barrier()

        pltpu.sync_copy(acc.at[pl.ds(base, rows_per_sub)],
                        out_hbm.at[c, pl.ds(base, rows_per_sub)])

    return seg_sum


def _make_count(n_pad, nblocks):
    mesh = plsc.VectorSubcoreMesh(core_axis_name="c", subcore_axis_name="s")
    rows_per_sub = n_pad // N_SUBCORES
    nchunks = nblocks // IDX_CHUNK
    scratch = [
        pltpu.VMEM((IDX_CHUNK, BATCH), jnp.int32),
        pltpu.VMEM((BATCH, 128), jnp.float32),
        pltpu.VMEM_SHARED((n_pad, 128), jnp.float32),
    ]

    @functools.partial(
        pl.kernel,
        out_type=jax.ShapeDtypeStruct((N_CORES, n_pad, 128), jnp.float32),
        mesh=mesh,
        scratch_types=scratch,
    )
    def count(dst_hbm, out_hbm, dst_v, ones_v, acc):
        c = lax.axis_index("c")
        s = lax.axis_index("s")
        wid = s * N_CORES + c

        _fill(ones_v, BATCH, 128, jnp.zeros((1, 16), jnp.float32))
        base = s * rows_per_sub
        _zero_acc_slice(ones_v, acc, base, rows_per_sub)
        _fill(ones_v, BATCH, 128, jnp.ones((1, 16), jnp.float32))
        plsc.subcore_barrier()

        @pl.loop(0, nchunks)
        def _(ci):
            pltpu.sync_copy(dst_hbm.at[wid, pl.ds(ci * IDX_CHUNK, IDX_CHUNK)],
                            dst_v)

            @pl.loop(0, IDX_CHUNK)
            def _(j):
                pltpu.sync_copy(ones_v, acc.at[dst_v.at[j]], add=True)

        plsc.subcore_barrier()
        pltpu.sync_copy(acc.at[pl.ds(base, rows_per_sub)],
                        out_hbm.at[c, pl.ds(base, rows_per_sub)])

    return count


_ROW_BLK = 400


def _mm2_kernel(x_ref, wa_ref, wb_ref, a_ref, b_ref):
    xb = x_ref[...]
    a_ref[...] = jnp.dot(xb, wa_ref[...], preferred_element_type=jnp.float32)
    b_ref[...] = jnp.dot(xb, wb_ref[...], preferred_element_type=jnp.float32)


def _mm2(x, wa_t, wb_t):
    n, d = x.shape
    grid = n // _ROW_BLK
    w_spec = pl.BlockSpec((d, wa_t.shape[1]), lambda i: (0, 0))
    row_spec = pl.BlockSpec((_ROW_BLK, d), lambda i: (i, 0))
    out_spec = pl.BlockSpec((_ROW_BLK, wa_t.shape[1]), lambda i: (i, 0))
    return pl.pallas_call(
        _mm2_kernel,
        grid=(grid,),
        in_specs=[row_spec, w_spec, w_spec],
        out_specs=[out_spec, out_spec],
        out_shape=[
            jax.ShapeDtypeStruct((n, wa_t.shape[1]), jnp.float32),
            jax.ShapeDtypeStruct((n, wb_t.shape[1]), jnp.float32),
        ],
    )(x, wa_t, wb_t)


def _combine_mm2_kernel(aggp_ref, cntp_ref, r_ref, bl_ref, wa_ref, wb_ref,
                        a_ref, b_ref):
    agg = aggp_ref[0] + aggp_ref[1]
    cnt = cntp_ref[0, :, 0:1] + cntp_ref[1, :, 0:1]
    mean = agg / jnp.maximum(cnt, 1.0)
    h = jnp.maximum(mean + bl_ref[...] + r_ref[...], 0.0)
    a_ref[...] = jnp.dot(h, wa_ref[...], preferred_element_type=jnp.float32)
    b_ref[...] = jnp.dot(h, wb_ref[...], preferred_element_type=jnp.float32)


def _combine_mm2(aggp, cntp, r, bl, wa_t, wb_t):
    n, d = r.shape
    grid = n // _ROW_BLK
    return pl.pallas_call(
        _combine_mm2_kernel,
        grid=(grid,),
        in_specs=[
            pl.BlockSpec((N_CORES, _ROW_BLK, d), lambda i: (0, i, 0)),
            pl.BlockSpec((N_CORES, _ROW_BLK, 128), lambda i: (0, i, 0)),
            pl.BlockSpec((_ROW_BLK, d), lambda i: (i, 0)),
            pl.BlockSpec((d,), lambda i: (0,)),
            pl.BlockSpec((d, wa_t.shape[1]), lambda i: (0, 0)),
            pl.BlockSpec((d, wb_t.shape[1]), lambda i: (0, 0)),
        ],
        out_specs=[
            pl.BlockSpec((_ROW_BLK, wa_t.shape[1]), lambda i: (i, 0)),
            pl.BlockSpec((_ROW_BLK, wb_t.shape[1]), lambda i: (i, 0)),
        ],
        out_shape=[
            jax.ShapeDtypeStruct((n, wa_t.shape[1]), jnp.float32),
            jax.ShapeDtypeStruct((n, wb_t.shape[1]), jnp.float32),
        ],
    )(aggp, cntp, r, bl, wa_t, wb_t)


def _combine_out_kernel(aggp_ref, cntp_ref, r_ref, bl_ref, w_ref, blin_ref,
                        o_ref):
    agg = aggp_ref[0] + aggp_ref[1]
    cnt = cntp_ref[0, :, 0:1] + cntp_ref[1, :, 0:1]
    mean = agg / jnp.maximum(cnt, 1.0)
    h = jnp.maximum(mean + bl_ref[...] + r_ref[...], 0.0)
    o_ref[...] = (
        jnp.dot(h, w_ref[...], preferred_element_type=jnp.float32)
        + blin_ref[...]
    )


def _combine_out(aggp, cntp, r, bl, w_t, blin):
    n, d = r.shape
    grid = n // _ROW_BLK
    return pl.pallas_call(
        _combine_out_kernel,
        grid=(grid,),
        in_specs=[
            pl.BlockSpec((N_CORES, _ROW_BLK, d), lambda i: (0, i, 0)),
            pl.BlockSpec((N_CORES, _ROW_BLK, 128), lambda i: (0, i, 0)),
            pl.BlockSpec((_ROW_BLK, d), lambda i: (i, 0)),
            pl.BlockSpec((d,), lambda i: (0,)),
            pl.BlockSpec((d, w_t.shape[1]), lambda i: (0, 0)),
            pl.BlockSpec((w_t.shape[1],), lambda i: (0,)),
        ],
        out_specs=pl.BlockSpec((_ROW_BLK, w_t.shape[1]), lambda i: (i, 0)),
        out_shape=jax.ShapeDtypeStruct((n, w_t.shape[1]), jnp.float32),
    )(aggp, cntp, r, bl, w_t, blin)


def kernel(x, edge_index, Wl1, bl1, Wr1, Wl2, bl2, Wr2, Wlin, blin):
    n, d = x.shape
    e = edge_index.shape[1]

    nblocks = -(-e // (N_WORKERS * BATCH * IDX_CHUNK)) * IDX_CHUNK
    e_pad = N_WORKERS * nblocks * BATCH
    n_pad = -(-(n + 1) // (N_SUBCORES * 8)) * (N_SUBCORES * 8)

    pad = e_pad - e
    src = jnp.concatenate(
        [edge_index[0], jnp.zeros((pad,), jnp.int32)]
    ).reshape(N_WORKERS, nblocks, BATCH)
    dst = jnp.concatenate(
        [edge_index[1], jnp.full((pad,), n, jnp.int32)]
    ).reshape(N_WORKERS, nblocks, BATCH)

    seg_sum = _make_seg_sum(n_pad, d, nblocks)
    count = _make_count(n_pad, nblocks)

    cntp = count(dst)
    y1, r1 = _mm2(x, Wl1.T, Wr1.T)
    aggp1 = seg_sum(y1, src, dst)
    y2, r2 = _combine_mm2(aggp1, cntp, r1, bl1, Wl2.T, Wr2.T)
    aggp2 = seg_sum(y2, src, dst)
    return _combine_out(aggp2, cntp, r2, bl2, Wlin.T, blin)
